# baseline (speedup 1.0000x reference)
import functools

import jax
import jax.numpy as jnp
from jax import lax
from jax.experimental import pallas as pl
from jax.experimental.pallas import tpu as pltpu

_H = 512
_W = 512
_BIG = _H + _W


def _tc_body(mask_ref, centers_ref, sizes_ref, bbox_ref):
    v = mask_ref[0]
    m = v > 0.5
    mf = m.astype(jnp.float32)

    count = jnp.sum(mf)
    yi = lax.broadcasted_iota(jnp.int32, (_H, _W), 0)
    xi = lax.broadcasted_iota(jnp.int32, (_H, _W), 1)
    yf = yi.astype(jnp.float32)
    xf = xi.astype(jnp.float32)

    sum_y = jnp.sum(yf * mf)
    sum_x = jnp.sum(xf * mf)

    big = jnp.int32(_BIG)
    y_min = jnp.min(jnp.where(m, yi, big))
    y_max = jnp.max(jnp.where(m, yi, jnp.int32(-1)))
    x_min = jnp.min(jnp.where(m, xi, big))
    x_max = jnp.max(jnp.where(m, xi, jnp.int32(-1)))

    denom = jnp.maximum(count, 1.0)
    center_y = sum_y / denom
    center_x = sum_x / denom

    height = y_max - y_min + 1
    width = x_max - x_min + 1
    size = jnp.maximum(height, width)

    cy_i = (y_min + y_max) // 2
    cx_i = (x_min + x_max) // 2
    half = size // 2
    y1 = jnp.maximum(0, cy_i - half)
    x1 = jnp.maximum(0, cx_i - half)
    y2 = jnp.minimum(_H, cy_i + half)
    x2 = jnp.minimum(_W, cx_i + half)

    empty = count == 0.0
    center_y = jnp.where(empty, jnp.float32(_H // 2), center_y)
    center_x = jnp.where(empty, jnp.float32(_W // 2), center_x)
    size_out = jnp.where(empty, jnp.int32(min(_H, _W) // 2), size)
    y1 = jnp.where(empty, jnp.int32(_H // 4), y1)
    x1 = jnp.where(empty, jnp.int32(_W // 4), x1)
    y2 = jnp.where(empty, jnp.int32(3 * _H // 4), y2)
    x2 = jnp.where(empty, jnp.int32(3 * _W // 4), x2)

    centers_ref[0, 0, 0] = center_y
    centers_ref[0, 0, 1] = center_x
    sizes_ref[0, 0, 0] = size_out
    bbox_ref[0, 0, 0] = y1
    bbox_ref[0, 0, 1] = x1
    bbox_ref[0, 0, 2] = y2
    bbox_ref[0, 0, 3] = x2


@jax.jit
def kernel(mask):
    B = mask.shape[0]
    m3 = mask.reshape(B, _H, _W)
    centers, sizes, bboxes = pl.pallas_call(
        _tc_body,
        grid=(B,),
        in_specs=[pl.BlockSpec((1, _H, _W), lambda b: (b, 0, 0))],
        out_specs=[
            pl.BlockSpec((1, 1, 2), lambda b: (b, 0, 0), memory_space=pltpu.SMEM),
            pl.BlockSpec((1, 1, 1), lambda b: (b, 0, 0), memory_space=pltpu.SMEM),
            pl.BlockSpec((1, 1, 4), lambda b: (b, 0, 0), memory_space=pltpu.SMEM),
        ],
        out_shape=[
            jax.ShapeDtypeStruct((B, 1, 2), jnp.float32),
            jax.ShapeDtypeStruct((B, 1, 1), jnp.int32),
            jax.ShapeDtypeStruct((B, 1, 4), jnp.int32),
        ],
    )(m3)
    return centers.reshape(B, 2), sizes.reshape(B), bboxes.reshape(B, 4)


# speedup vs baseline: 1.1801x; 1.1801x over previous
import functools

import jax
import jax.numpy as jnp
from jax import lax
from jax.experimental import pallas as pl
from jax.experimental.pallas import tpu as pltpu

_H = 512
_W = 512
_BIG = _H + _W


def _tc_body(mask_ref, centers_ref, sizes_ref, bbox_ref):
    v = mask_ref[0]
    mf = (v > 0.5).astype(jnp.float32)

    ones_w = jnp.ones((_W, 1), jnp.float32)
    rowcount = jax.lax.dot_general(
        mf, ones_w, (((1,), (0,)), ((), ())),
        preferred_element_type=jnp.float32)[:, 0]
    colcount = jnp.sum(mf, axis=0)

    yidx = lax.iota(jnp.int32, _H).astype(jnp.float32)
    xidx = lax.iota(jnp.int32, _W).astype(jnp.float32)
    count = jnp.sum(rowcount)
    sum_y = jnp.sum(yidx * rowcount)
    sum_x = jnp.sum(xidx * colcount)

    bigf = jnp.float32(_BIG)
    y_min = jnp.min(jnp.where(rowcount > 0.0, yidx, bigf)).astype(jnp.int32)
    y_max = jnp.max(jnp.where(rowcount > 0.0, yidx, -1.0)).astype(jnp.int32)
    x_min = jnp.min(jnp.where(colcount > 0.0, xidx, bigf)).astype(jnp.int32)
    x_max = jnp.max(jnp.where(colcount > 0.0, xidx, -1.0)).astype(jnp.int32)

    denom = jnp.maximum(count, 1.0)
    center_y = sum_y / denom
    center_x = sum_x / denom

    height = y_max - y_min + 1
    width = x_max - x_min + 1
    size = jnp.maximum(height, width)

    cy_i = (y_min + y_max) // 2
    cx_i = (x_min + x_max) // 2
    half = size // 2
    y1 = jnp.maximum(0, cy_i - half)
    x1 = jnp.maximum(0, cx_i - half)
    y2 = jnp.minimum(_H, cy_i + half)
    x2 = jnp.minimum(_W, cx_i + half)

    empty = count == 0.0
    center_y = jnp.where(empty, jnp.float32(_H // 2), center_y)
    center_x = jnp.where(empty, jnp.float32(_W // 2), center_x)
    size_out = jnp.where(empty, jnp.int32(min(_H, _W) // 2), size)
    y1 = jnp.where(empty, jnp.int32(_H // 4), y1)
    x1 = jnp.where(empty, jnp.int32(_W // 4), x1)
    y2 = jnp.where(empty, jnp.int32(3 * _H // 4), y2)
    x2 = jnp.where(empty, jnp.int32(3 * _W // 4), x2)

    centers_ref[0, 0, 0] = center_y
    centers_ref[0, 0, 1] = center_x
    sizes_ref[0, 0, 0] = size_out
    bbox_ref[0, 0, 0] = y1
    bbox_ref[0, 0, 1] = x1
    bbox_ref[0, 0, 2] = y2
    bbox_ref[0, 0, 3] = x2


@jax.jit
def kernel(mask):
    B = mask.shape[0]
    m3 = mask.reshape(B, _H, _W)
    centers, sizes, bboxes = pl.pallas_call(
        _tc_body,
        grid=(B,),
        in_specs=[pl.BlockSpec((1, _H, _W), lambda b: (b, 0, 0))],
        out_specs=[
            pl.BlockSpec((1, 1, 2), lambda b: (b, 0, 0), memory_space=pltpu.SMEM),
            pl.BlockSpec((1, 1, 1), lambda b: (b, 0, 0), memory_space=pltpu.SMEM),
            pl.BlockSpec((1, 1, 4), lambda b: (b, 0, 0), memory_space=pltpu.SMEM),
        ],
        out_shape=[
            jax.ShapeDtypeStruct((B, 1, 2), jnp.float32),
            jax.ShapeDtypeStruct((B, 1, 1), jnp.int32),
            jax.ShapeDtypeStruct((B, 1, 4), jnp.int32),
        ],
    )(m3)
    return centers.reshape(B, 2), sizes.reshape(B), bboxes.reshape(B, 4)
